# baseline (speedup 1.0000x reference)
import functools

import jax
import jax.numpy as jnp
from jax import lax
from jax.experimental import pallas as pl
from jax.experimental.pallas import tpu as pltpu
from jax.experimental.pallas import tpu_sc as plsc

N = 10000
E = 320000
EL = 20000
H = 128

NC = 2
NS = 16
NW = NC * NS
EPW = E // NW
ECH = 128
NFULL = EPW // ECH
ETAIL = EPW - NFULL * ECH
RPS = N // NS
NP = 10240

_mesh = plsc.VectorSubcoreMesh(core_axis_name="c", subcore_axis_name="s")


@functools.partial(
    pl.kernel,
    mesh=_mesh,
    out_type=jax.ShapeDtypeStruct((NC, NP), jnp.float32),
    scratch_types=[
        pltpu.VMEM_SHARED((NP,), jnp.float32),
        pltpu.VMEM((3, ECH), jnp.int32),
        pltpu.VMEM((ETAIL,), jnp.int32),
        pltpu.VMEM((ECH,), jnp.float32),
        pltpu.VMEM((640,), jnp.float32),
        pltpu.SemaphoreType.DMA,
        pltpu.SemaphoreType.DMA,
        pltpu.SemaphoreType.DMA,
        pltpu.SemaphoreType.DMA,
        pltpu.SemaphoreType.DMA,
        pltpu.SemaphoreType.DMA,
    ],
)
def _deg_kernel(dst_hbm, cnt_hbm, acc, didx3, didx_t, ones_v, zbuf,
                ss0, ss1, ss2, sid0, sid1, sid2):
    c = lax.axis_index("c")
    s = lax.axis_index("s")
    wid = c * NS + s
    ss = (ss0, ss1, ss2)
    sid = (sid0, sid1, sid2)

    def zfill(i, carry):
        zbuf[pl.ds(i * 16, 16)] = jnp.zeros((16,), jnp.float32)
        return carry

    lax.fori_loop(0, 640 // 16, zfill, None)

    pltpu.sync_copy(zbuf, acc.at[pl.ds(s * 640, 640)])

    def fill(i, carry):
        ones_v[pl.ds(i * 16, 16)] = jnp.ones((16,), jnp.float32)
        return carry

    lax.fori_loop(0, ECH // 16, fill, None)
    plsc.subcore_barrier()

    base = wid * EPW

    def issue_idx(j, a):
        pltpu.async_copy(dst_hbm.at[pl.ds(base + j * ECH, ECH)],
                         didx3.at[a], sid[a])

    def wait_idx(sem):
        pltpu.make_async_copy(dst_hbm.at[pl.ds(base, ECH)],
                              didx3.at[0], sem).wait()

    def issue_scat(a):
        pltpu.async_copy(ones_v, acc.at[didx3.at[a]], ss[a], add=True)

    def wait_scat(sem):
        pltpu.make_async_copy(ones_v, acc.at[didx3.at[0]], sem).wait()

    issue_idx(0, 0)
    issue_idx(1, 1)

    def position(j, a, first):
        cslot = (a + 2) % 3
        wait_idx(sid[a])
        issue_scat(a)
        if not first:
            wait_scat(ss[cslot])
        issue_idx(j + 2, cslot)

    def body(jj, carry):
        j0 = 3 * jj

        @pl.when(jj == 0)
        def _():
            position(j0, 0, True)

        @pl.when(jj > 0)
        def _():
            position(j0, 0, False)

        position(j0 + 1, 1, False)
        position(j0 + 2, 2, False)
        return carry

    lax.fori_loop(0, NFULL // 3, body, None)

    wait_scat(ss[2])
    wait_idx(sid[0])
    wait_idx(sid[1])

    pltpu.sync_copy(dst_hbm.at[pl.ds(base + NFULL * ECH, ETAIL)], didx_t)
    pltpu.sync_copy(ones_v.at[pl.ds(0, ETAIL)], acc.at[didx_t], add=True)
    plsc.subcore_barrier()

    pltpu.sync_copy(acc.at[pl.ds(s * 640, 640)], zbuf)
    pltpu.sync_copy(zbuf, cnt_hbm.at[c, pl.ds(s * 640, 640)])


ECB = 2500 // NW
EPW2 = ECB * ECH
XCH = 2500 - ECB * NW


@functools.partial(
    pl.kernel,
    mesh=_mesh,
    out_type=jax.ShapeDtypeStruct((NC, N, H), jnp.float32),
    scratch_types=[
        pltpu.VMEM_SHARED((N, H), jnp.float32),
        pltpu.VMEM((3, ECH), jnp.int32),
        pltpu.VMEM((3, ECH), jnp.int32),
        pltpu.VMEM((ECH, H), jnp.float32),
        pltpu.VMEM((ECH, H), jnp.float32),
        pltpu.VMEM((ECH, H), jnp.float32),
        pltpu.VMEM((ECH,), jnp.int32),
        pltpu.SemaphoreType.DMA,
        pltpu.SemaphoreType.DMA,
        pltpu.SemaphoreType.DMA,
        pltpu.SemaphoreType.DMA,
        pltpu.SemaphoreType.DMA,
        pltpu.SemaphoreType.DMA,
        pltpu.SemaphoreType.DMA,
        pltpu.SemaphoreType.DMA,
        pltpu.SemaphoreType.DMA,
        pltpu.SemaphoreType.DMA,
        pltpu.SemaphoreType.DMA,
        pltpu.SemaphoreType.DMA,
    ],
)
def _agg_kernel(t_hbm, src_hbm, dst_hbm, zrows_hbm, p_hbm,
                acc, sidx3, didx3, rows0, rows1, rows2, sidx_t,
                sg0, sg1, sg2, ss0, ss1, ss2,
                sis0, sis1, sis2, sid0, sid1, sid2):
    c = lax.axis_index("c")
    s = lax.axis_index("s")
    wid = c * NS + s
    rows = (rows0, rows1, rows2)
    sg = (sg0, sg1, sg2)
    ss = (ss0, ss1, ss2)
    sis = (sis0, sis1, sis2)
    sid = (sid0, sid1, sid2)

    @pl.when(s < NS - 1)
    def _():
        pltpu.sync_copy(zrows_hbm.at[pl.ds(s * 624, 624)],
                        acc.at[pl.ds(s * 624, 624)])

    @pl.when(s == NS - 1)
    def _():
        pltpu.sync_copy(zrows_hbm.at[pl.ds(9360, 640)],
                        acc.at[pl.ds(9360, 640)])

    plsc.subcore_barrier()

    base = wid * EPW2

    def issue_idx_src(j, a):
        pltpu.async_copy(src_hbm.at[pl.ds(base + j * ECH, ECH)],
                         sidx3.at[a], sis[a])

    def issue_idx_dst(j, a):
        pltpu.async_copy(dst_hbm.at[pl.ds(base + j * ECH, ECH)],
                         didx3.at[a], sid[a])

    def wait_idx(sem):
        pltpu.make_async_copy(src_hbm.at[pl.ds(base, ECH)],
                              sidx3.at[0], sem).wait()

    def issue_gather(a):
        pltpu.async_copy(t_hbm.at[sidx3.at[a]], rows[a], sg[a])

    def wait_gather(a):
        pltpu.make_async_copy(t_hbm.at[sidx3.at[0]], rows[0], sg[a]).wait()

    def issue_scat(a):
        pltpu.async_copy(rows[a], acc.at[didx3.at[a]], ss[a], add=True)

    def wait_scat(sem):
        pltpu.make_async_copy(rows[0], acc.at[didx3.at[0]], sem).wait()

    for a in range(3):
        issue_idx_src(a, a)
    issue_idx_dst(0, 0)
    issue_idx_dst(1, 1)
    for a in range(2):
        wait_idx(sis[a])
        issue_gather(a)

    def position(j, a, first):
        cslot = (a + 2) % 3
        wait_gather(a)
        issue_idx_src(j + 3, a)
        wait_idx(sid[a])
        issue_scat(a)
        if not first:
            wait_scat(ss[cslot])
        issue_idx_dst(j + 2, cslot)
        wait_idx(sis[cslot])
        issue_gather(cslot)

    def body(jj, carry):
        j0 = 3 * jj

        @pl.when(jj == 0)
        def _():
            position(j0, 0, True)

        @pl.when(jj > 0)
        def _():
            position(j0, 0, False)

        position(j0 + 1, 1, False)
        position(j0 + 2, 2, False)
        return carry

    lax.fori_loop(0, ECB // 3, body, None)

    wait_scat(ss[2])
    wait_gather(0)
    wait_gather(1)
    wait_idx(sis[2])
    wait_idx(sid[0])
    wait_idx(sid[1])

    @pl.when(wid < XCH)
    def _():
        xbase = (NW * ECB + wid) * ECH
        pltpu.sync_copy(src_hbm.at[pl.ds(xbase, ECH)], sidx_t)
        pltpu.sync_copy(dst_hbm.at[pl.ds(xbase, ECH)], didx3.at[0])
        pltpu.async_copy(t_hbm.at[sidx_t], rows0, sg0).wait()
        pltpu.sync_copy(rows0, acc.at[didx3.at[0]], add=True)

    plsc.subcore_barrier()

    @pl.when(s < NS - 1)
    def _():
        pltpu.sync_copy(acc.at[pl.ds(s * 624, 624)],
                        p_hbm.at[c, pl.ds(s * 624, 624)])

    @pl.when(s == NS - 1)
    def _():
        pltpu.sync_copy(acc.at[pl.ds(9360, 640)],
                        p_hbm.at[c, pl.ds(9360, 640)])


DCH = 80
KPW = 8
EL_OUT = NW * KPW * DCH
EL_IDX = NW * (KPW + 2) * DCH


@functools.partial(
    pl.kernel,
    mesh=_mesh,
    out_type=jax.ShapeDtypeStruct((EL_OUT, H), jnp.float32),
    scratch_types=[
        pltpu.VMEM((2, DCH), jnp.int32),
        pltpu.VMEM((2, DCH), jnp.int32),
        pltpu.VMEM((DCH, H), jnp.float32),
        pltpu.VMEM((DCH, H), jnp.float32),
        pltpu.VMEM((DCH, H), jnp.float32),
        pltpu.VMEM((DCH, H), jnp.float32),
        pltpu.VMEM((DCH, H), jnp.float32),
        pltpu.VMEM((DCH, H), jnp.float32),
        pltpu.SemaphoreType.DMA,
        pltpu.SemaphoreType.DMA,
        pltpu.SemaphoreType.DMA,
        pltpu.SemaphoreType.DMA,
        pltpu.SemaphoreType.DMA,
        pltpu.SemaphoreType.DMA,
    ],
)
def _dec_kernel(emb_hbm, els_hbm, eld_hbm, out_hbm,
                sidx2, didx2, rs0, rs1, rd0, rd1, ob0, ob1,
                si0, si1, sg0, sg1, sw0, sw1):
    c = lax.axis_index("c")
    s = lax.axis_index("s")
    wid = c * NS + s
    rs = (rs0, rs1)
    rd = (rd0, rd1)
    ob = (ob0, ob1)
    si = (si0, si1)
    sg = (sg0, sg1)
    sw = (sw0, sw1)

    def goff(k):
        return (wid + k * NW) * DCH

    def issue_idx(k, b):
        pltpu.async_copy(els_hbm.at[pl.ds(goff(k), DCH)], sidx2.at[b], si[b])
        pltpu.async_copy(eld_hbm.at[pl.ds(goff(k), DCH)], didx2.at[b], si[b])

    def wait_idx(b):
        for _ in range(2):
            pltpu.make_async_copy(els_hbm.at[pl.ds(goff(0), DCH)],
                                  sidx2.at[0], si[b]).wait()

    def issue_gathers(b):
        pltpu.async_copy(emb_hbm.at[sidx2.at[b]], rs[b], sg[b])
        pltpu.async_copy(emb_hbm.at[didx2.at[b]], rd[b], sg[b])

    def wait_gathers(b):
        for _ in range(2):
            pltpu.make_async_copy(emb_hbm.at[sidx2.at[0]], rs[0],
                                  sg[b]).wait()

    def wait_wb(b):
        pltpu.make_async_copy(ob[0], out_hbm.at[pl.ds(goff(0), DCH)],
                              sw[b]).wait()

    issue_idx(0, 0)
    issue_idx(1, 1)
    wait_idx(0)
    issue_gathers(0)

    def position(k, b, first):
        o = 1 - b
        wait_gathers(b)
        issue_idx(k + 2, b)
        wait_idx(o)
        issue_gathers(o)
        if not first:
            wait_wb(b)

        def pair(p, gcarry):
            for j in range(H // 16):
                a = rs[b][p, pl.ds(j * 16, 16)]
                d = rd[b][p, pl.ds(j * 16, 16)]
                ob[b][p, pl.ds(j * 16, 16)] = a * d
            return gcarry

        lax.fori_loop(0, DCH, pair, None)
        pltpu.async_copy(ob[b], out_hbm.at[pl.ds(goff(k), DCH)], sw[b])

    def body(kk, carry):
        k0 = 2 * kk

        @pl.when(kk == 0)
        def _():
            position(k0, 0, True)
            position(k0 + 1, 1, True)

        @pl.when(kk > 0)
        def _():
            position(k0, 0, False)
            position(k0 + 1, 1, False)

        return carry

    lax.fori_loop(0, KPW // 2, body, None)

    wait_gathers(0)
    wait_idx(1)
    wait_wb(0)
    wait_wb(1)


RB = 1000
GRID = N // RB

_row_spec = pl.BlockSpec((RB, H), lambda i: (i, 0))
_w_spec = pl.BlockSpec((H, H), lambda i: (0, 0))
_b_spec = pl.BlockSpec((1, H), lambda i: (0, 0))
_cnt_spec = pl.BlockSpec((RB, 2), lambda i: (i, 0))
_p_spec = pl.BlockSpec((2, RB, H), lambda i: (0, i, 0))
_wp_spec = pl.BlockSpec((8, H), lambda i: (0, 0))


def _dinv_of(cnt_blk):
    return (1.0 / jnp.sqrt(jnp.sum(cnt_blk, axis=1) + 1.0))[:, None]


def _tc_a_body(x_ref, wpre_ref, bpre_ref, wg0_ref, cnt_ref, t0_ref):
    h = jnp.maximum(
        jnp.dot(x_ref[...], wpre_ref[...], preferred_element_type=jnp.float32)
        + bpre_ref[...], 0.0)
    hw = jnp.dot(h, wg0_ref[...], preferred_element_type=jnp.float32)
    t0_ref[...] = hw * _dinv_of(cnt_ref[...])


def _tc_b_body(p_ref, t0_ref, cnt_ref, bg0_ref, wg1_ref, emb0_ref, t1_ref):
    dinv = _dinv_of(cnt_ref[...])
    emb0 = jnp.maximum(
        (p_ref[0] + p_ref[1] + t0_ref[...]) * dinv + bg0_ref[...], 0.0)
    emb0_ref[...] = emb0
    t1_ref[...] = jnp.dot(
        emb0, wg1_ref[...], preferred_element_type=jnp.float32) * dinv


def _tc_c_body(p_ref, t1_ref, cnt_ref, bg1_ref, emb1_ref):
    dinv = _dinv_of(cnt_ref[...])
    emb1_ref[...] = jnp.maximum(
        (p_ref[0] + p_ref[1] + t1_ref[...]) * dinv + bg1_ref[...], 0.0)


RBD = 2000


def _tc_d_body(hh_ref, wpost_ref, bpost_ref, sc_ref):
    logits = jnp.dot(hh_ref[...], wpost_ref[...]) + bpost_ref[...]
    sc_ref[...] = jnp.sum(logits, axis=-1)


_tc_a = pl.pallas_call(
    _tc_a_body,
    grid=(GRID,),
    in_specs=[_row_spec, _w_spec, _b_spec, _w_spec, _cnt_spec],
    out_specs=_row_spec,
    out_shape=jax.ShapeDtypeStruct((N, H), jnp.float32),
)

_tc_b = pl.pallas_call(
    _tc_b_body,
    grid=(GRID,),
    in_specs=[_p_spec, _row_spec, _cnt_spec, _b_spec, _w_spec],
    out_specs=[_row_spec, _row_spec],
    out_shape=[jax.ShapeDtypeStruct((N, H), jnp.float32),
               jax.ShapeDtypeStruct((N, H), jnp.float32)],
)

_tc_c = pl.pallas_call(
    _tc_c_body,
    grid=(GRID,),
    in_specs=[_p_spec, _row_spec, _cnt_spec, _b_spec],
    out_specs=_row_spec,
    out_shape=jax.ShapeDtypeStruct((N, H), jnp.float32),
)

_tc_d = pl.pallas_call(
    _tc_d_body,
    out_shape=jax.ShapeDtypeStruct((EL_OUT,), jnp.float32),
)


def kernel(x, edge_index, edge_label_index, W_pre, b_pre, W_g0, b_g0,
           W_g1, b_g1, W_post, b_post):
    f32 = jnp.float32
    src = edge_index[0]
    dst = edge_index[1]
    pad = jnp.zeros((EL_IDX - EL,), edge_label_index.dtype)
    els = jnp.concatenate([edge_label_index[0], pad])
    eld = jnp.concatenate([edge_label_index[1], pad])
    zeros_nh = jnp.zeros((N, H), f32)
    b_pre2 = b_pre.reshape(1, H)
    bg02 = b_g0.reshape(1, H)
    bg12 = b_g1.reshape(1, H)

    cnt = _deg_kernel(dst)
    cnt_t = cnt[:, :N].T

    t0 = _tc_a(x, W_pre, b_pre2, W_g0, cnt_t)
    p0 = _agg_kernel(t0, src, dst, zeros_nh)
    emb0, t1 = _tc_b(p0, t0, cnt_t, bg02, W_g1)
    p1 = _agg_kernel(t1, src, dst, zeros_nh)
    emb1 = _tc_c(p1, t1, cnt_t, bg12)
    hh = _dec_kernel(emb1, els, eld)
    scores = _tc_d(hh, W_post, b_post.reshape(1, 2))[:EL]
    return scores, emb0, emb1


# speedup vs baseline: 1.6836x; 1.6836x over previous
import functools

import jax
import jax.numpy as jnp
from jax import lax
from jax.experimental import pallas as pl
from jax.experimental.pallas import tpu as pltpu
from jax.experimental.pallas import tpu_sc as plsc

N = 10000
E = 320000
EL = 20000
H = 128

NC = 2
NS = 16
NW = NC * NS
EPW = E // NW
ECH = 128
NFULL = EPW // ECH
ETAIL = EPW - NFULL * ECH
RPS = N // NS
NP = 10240

_mesh = plsc.VectorSubcoreMesh(core_axis_name="c", subcore_axis_name="s")


@functools.partial(
    pl.kernel,
    mesh=_mesh,
    out_type=jax.ShapeDtypeStruct((NC, NP), jnp.float32),
    scratch_types=[
        pltpu.VMEM_SHARED((NP,), jnp.float32),
        pltpu.VMEM((3, ECH), jnp.int32),
        pltpu.VMEM((ETAIL,), jnp.int32),
        pltpu.VMEM((ECH,), jnp.float32),
        pltpu.VMEM((640,), jnp.float32),
        pltpu.SemaphoreType.DMA,
        pltpu.SemaphoreType.DMA,
        pltpu.SemaphoreType.DMA,
        pltpu.SemaphoreType.DMA,
        pltpu.SemaphoreType.DMA,
        pltpu.SemaphoreType.DMA,
    ],
)
def _deg_kernel(dst_hbm, cnt_hbm, acc, didx3, didx_t, ones_v, zbuf,
                ss0, ss1, ss2, sid0, sid1, sid2):
    c = lax.axis_index("c")
    s = lax.axis_index("s")
    wid = c * NS + s
    ss = (ss0, ss1, ss2)
    sid = (sid0, sid1, sid2)

    def zfill(i, carry):
        zbuf[pl.ds(i * 16, 16)] = jnp.zeros((16,), jnp.float32)
        return carry

    lax.fori_loop(0, 640 // 16, zfill, None)

    pltpu.sync_copy(zbuf, acc.at[pl.ds(s * 640, 640)])

    def fill(i, carry):
        ones_v[pl.ds(i * 16, 16)] = jnp.ones((16,), jnp.float32)
        return carry

    lax.fori_loop(0, ECH // 16, fill, None)
    plsc.subcore_barrier()

    base = wid * EPW

    def issue_idx(j, a):
        pltpu.async_copy(dst_hbm.at[pl.ds(base + j * ECH, ECH)],
                         didx3.at[a], sid[a])

    def wait_idx(sem):
        pltpu.make_async_copy(dst_hbm.at[pl.ds(base, ECH)],
                              didx3.at[0], sem).wait()

    def issue_scat(a):
        pltpu.async_copy(ones_v, acc.at[didx3.at[a]], ss[a], add=True)

    def wait_scat(sem):
        pltpu.make_async_copy(ones_v, acc.at[didx3.at[0]], sem).wait()

    issue_idx(0, 0)
    issue_idx(1, 1)

    def position(j, a, first):
        cslot = (a + 2) % 3
        wait_idx(sid[a])
        issue_scat(a)
        if not first:
            wait_scat(ss[cslot])
        issue_idx(j + 2, cslot)

    def body(jj, carry):
        j0 = 3 * jj

        @pl.when(jj == 0)
        def _():
            position(j0, 0, True)

        @pl.when(jj > 0)
        def _():
            position(j0, 0, False)

        position(j0 + 1, 1, False)
        position(j0 + 2, 2, False)
        return carry

    lax.fori_loop(0, NFULL // 3, body, None)

    wait_scat(ss[2])
    wait_idx(sid[0])
    wait_idx(sid[1])

    pltpu.sync_copy(dst_hbm.at[pl.ds(base + NFULL * ECH, ETAIL)], didx_t)
    pltpu.sync_copy(ones_v.at[pl.ds(0, ETAIL)], acc.at[didx_t], add=True)
    plsc.subcore_barrier()

    pltpu.sync_copy(acc.at[pl.ds(s * 640, 640)], zbuf)
    pltpu.sync_copy(zbuf, cnt_hbm.at[c, pl.ds(s * 640, 640)])


ECB = 2500 // NW
EPW2 = ECB * ECH
XCH = 2500 - ECB * NW


@functools.partial(
    pl.kernel,
    mesh=_mesh,
    out_type=jax.ShapeDtypeStruct((NC, N, H), jnp.float32),
    scratch_types=[
        pltpu.VMEM_SHARED((N, H), jnp.float32),
        pltpu.VMEM((3, ECH), jnp.int32),
        pltpu.VMEM((3, ECH), jnp.int32),
        pltpu.VMEM((ECH, H), jnp.float32),
        pltpu.VMEM((ECH, H), jnp.float32),
        pltpu.VMEM((ECH, H), jnp.float32),
        pltpu.VMEM((ECH,), jnp.int32),
        pltpu.SemaphoreType.DMA,
        pltpu.SemaphoreType.DMA,
        pltpu.SemaphoreType.DMA,
        pltpu.SemaphoreType.DMA,
        pltpu.SemaphoreType.DMA,
        pltpu.SemaphoreType.DMA,
        pltpu.SemaphoreType.DMA,
        pltpu.SemaphoreType.DMA,
        pltpu.SemaphoreType.DMA,
        pltpu.SemaphoreType.DMA,
        pltpu.SemaphoreType.DMA,
        pltpu.SemaphoreType.DMA,
    ],
)
def _agg_kernel(t_hbm, src_hbm, dst_hbm, zrows_hbm, p_hbm,
                acc, sidx3, didx3, rows0, rows1, rows2, sidx_t,
                sg0, sg1, sg2, ss0, ss1, ss2,
                sis0, sis1, sis2, sid0, sid1, sid2):
    c = lax.axis_index("c")
    s = lax.axis_index("s")
    wid = c * NS + s
    rows = (rows0, rows1, rows2)
    sg = (sg0, sg1, sg2)
    ss = (ss0, ss1, ss2)
    sis = (sis0, sis1, sis2)
    sid = (sid0, sid1, sid2)

    @pl.when(s < NS - 1)
    def _():
        pltpu.sync_copy(zrows_hbm.at[pl.ds(s * 624, 624)],
                        acc.at[pl.ds(s * 624, 624)])

    @pl.when(s == NS - 1)
    def _():
        pltpu.sync_copy(zrows_hbm.at[pl.ds(9360, 640)],
                        acc.at[pl.ds(9360, 640)])

    plsc.subcore_barrier()

    base = wid * EPW2

    def issue_idx_src(j, a):
        pltpu.async_copy(src_hbm.at[pl.ds(base + j * ECH, ECH)],
                         sidx3.at[a], sis[a])

    def issue_idx_dst(j, a):
        pltpu.async_copy(dst_hbm.at[pl.ds(base + j * ECH, ECH)],
                         didx3.at[a], sid[a])

    def wait_idx(sem):
        pltpu.make_async_copy(src_hbm.at[pl.ds(base, ECH)],
                              sidx3.at[0], sem).wait()

    def issue_gather(a):
        pltpu.async_copy(t_hbm.at[sidx3.at[a]], rows[a], sg[a])

    def wait_gather(a):
        pltpu.make_async_copy(t_hbm.at[sidx3.at[0]], rows[0], sg[a]).wait()

    def issue_scat(a):
        pltpu.async_copy(rows[a], acc.at[didx3.at[a]], ss[a], add=True)

    def wait_scat(sem):
        pltpu.make_async_copy(rows[0], acc.at[didx3.at[0]], sem).wait()

    for a in range(3):
        issue_idx_src(a, a)
    issue_idx_dst(0, 0)
    issue_idx_dst(1, 1)
    for a in range(2):
        wait_idx(sis[a])
        issue_gather(a)

    def position(j, a, first):
        cslot = (a + 2) % 3
        wait_gather(a)
        issue_idx_src(j + 3, a)
        wait_idx(sid[a])
        issue_scat(a)
        if not first:
            wait_scat(ss[cslot])
        issue_idx_dst(j + 2, cslot)
        wait_idx(sis[cslot])
        issue_gather(cslot)

    def body(jj, carry):
        j0 = 3 * jj

        @pl.when(jj == 0)
        def _():
            position(j0, 0, True)

        @pl.when(jj > 0)
        def _():
            position(j0, 0, False)

        position(j0 + 1, 1, False)
        position(j0 + 2, 2, False)
        return carry

    lax.fori_loop(0, ECB // 3, body, None)

    wait_scat(ss[2])
    wait_gather(0)
    wait_gather(1)
    wait_idx(sis[2])
    wait_idx(sid[0])
    wait_idx(sid[1])

    @pl.when(wid < XCH)
    def _():
        xbase = (NW * ECB + wid) * ECH
        pltpu.sync_copy(src_hbm.at[pl.ds(xbase, ECH)], sidx_t)
        pltpu.sync_copy(dst_hbm.at[pl.ds(xbase, ECH)], didx3.at[0])
        pltpu.async_copy(t_hbm.at[sidx_t], rows0, sg0).wait()
        pltpu.sync_copy(rows0, acc.at[didx3.at[0]], add=True)

    plsc.subcore_barrier()

    @pl.when(s < NS - 1)
    def _():
        pltpu.sync_copy(acc.at[pl.ds(s * 624, 624)],
                        p_hbm.at[c, pl.ds(s * 624, 624)])

    @pl.when(s == NS - 1)
    def _():
        pltpu.sync_copy(acc.at[pl.ds(9360, 640)],
                        p_hbm.at[c, pl.ds(9360, 640)])


DCH = 80
KPW = 8
EL_OUT = NW * KPW * DCH
EL_IDX = NW * (KPW + 2) * DCH


@functools.partial(
    pl.kernel,
    mesh=_mesh,
    out_type=jax.ShapeDtypeStruct((EL_OUT, H), jnp.float32),
    scratch_types=[
        pltpu.VMEM((2, DCH), jnp.int32),
        pltpu.VMEM((2, DCH), jnp.int32),
        pltpu.VMEM((DCH, H), jnp.float32),
        pltpu.VMEM((DCH, H), jnp.float32),
        pltpu.VMEM((DCH, H), jnp.float32),
        pltpu.VMEM((DCH, H), jnp.float32),
        pltpu.VMEM((DCH, H), jnp.float32),
        pltpu.VMEM((DCH, H), jnp.float32),
        pltpu.SemaphoreType.DMA,
        pltpu.SemaphoreType.DMA,
        pltpu.SemaphoreType.DMA,
        pltpu.SemaphoreType.DMA,
        pltpu.SemaphoreType.DMA,
        pltpu.SemaphoreType.DMA,
    ],
)
def _dec_kernel(emb_hbm, els_hbm, eld_hbm, out_hbm,
                sidx2, didx2, rs0, rs1, rd0, rd1, ob0, ob1,
                si0, si1, sg0, sg1, sw0, sw1):
    c = lax.axis_index("c")
    s = lax.axis_index("s")
    wid = c * NS + s
    rs = (rs0, rs1)
    rd = (rd0, rd1)
    ob = (ob0, ob1)
    si = (si0, si1)
    sg = (sg0, sg1)
    sw = (sw0, sw1)

    def goff(k):
        return (wid + k * NW) * DCH

    def issue_idx(k, b):
        pltpu.async_copy(els_hbm.at[pl.ds(goff(k), DCH)], sidx2.at[b], si[b])
        pltpu.async_copy(eld_hbm.at[pl.ds(goff(k), DCH)], didx2.at[b], si[b])

    def wait_idx(b):
        for _ in range(2):
            pltpu.make_async_copy(els_hbm.at[pl.ds(goff(0), DCH)],
                                  sidx2.at[0], si[b]).wait()

    def issue_gathers(b):
        pltpu.async_copy(emb_hbm.at[sidx2.at[b]], rs[b], sg[b])
        pltpu.async_copy(emb_hbm.at[didx2.at[b]], rd[b], sg[b])

    def wait_gathers(b):
        for _ in range(2):
            pltpu.make_async_copy(emb_hbm.at[sidx2.at[0]], rs[0],
                                  sg[b]).wait()

    def wait_wb(b):
        pltpu.make_async_copy(ob[0], out_hbm.at[pl.ds(goff(0), DCH)],
                              sw[b]).wait()

    issue_idx(0, 0)
    issue_idx(1, 1)
    wait_idx(0)
    issue_gathers(0)

    def position(k, b, first):
        o = 1 - b
        wait_gathers(b)
        issue_idx(k + 2, b)
        wait_idx(o)
        issue_gathers(o)
        if not first:
            wait_wb(b)

        def pair(p, gcarry):
            for j in range(H // 16):
                a = rs[b][p, pl.ds(j * 16, 16)]
                d = rd[b][p, pl.ds(j * 16, 16)]
                ob[b][p, pl.ds(j * 16, 16)] = a * d
            return gcarry

        lax.fori_loop(0, DCH, pair, None)
        pltpu.async_copy(ob[b], out_hbm.at[pl.ds(goff(k), DCH)], sw[b])

    def body(kk, carry):
        k0 = 2 * kk

        @pl.when(kk == 0)
        def _():
            position(k0, 0, True)
            position(k0 + 1, 1, True)

        @pl.when(kk > 0)
        def _():
            position(k0, 0, False)
            position(k0 + 1, 1, False)

        return carry

    lax.fori_loop(0, KPW // 2, body, None)

    wait_gathers(0)
    wait_idx(1)
    wait_wb(0)
    wait_wb(1)


RB = 1000
GRID = N // RB

_row_spec = pl.BlockSpec((RB, H), lambda i: (i, 0))
_w_spec = pl.BlockSpec((H, H), lambda i: (0, 0))
_b_spec = pl.BlockSpec((1, H), lambda i: (0, 0))
_cnt_spec = pl.BlockSpec((RB, 2), lambda i: (i, 0))
_p_spec = pl.BlockSpec((2, RB, H), lambda i: (0, i, 0))
_wp_spec = pl.BlockSpec((8, H), lambda i: (0, 0))


def _dinv_of(cnt_blk):
    return (1.0 / jnp.sqrt(jnp.sum(cnt_blk, axis=1) + 1.0))[:, None]


def _tc_a_body(x_ref, wpre_ref, bpre_ref, wg0_ref, cnt_ref, t0_ref):
    h = jnp.maximum(
        jnp.dot(x_ref[...], wpre_ref[...], preferred_element_type=jnp.float32)
        + bpre_ref[...], 0.0)
    hw = jnp.dot(h, wg0_ref[...], preferred_element_type=jnp.float32)
    t0_ref[...] = hw * _dinv_of(cnt_ref[...])


def _tc_b_body(p_ref, t0_ref, cnt_ref, bg0_ref, wg1_ref, emb0_ref, t1_ref):
    dinv = _dinv_of(cnt_ref[...])
    emb0 = jnp.maximum(
        (p_ref[0] + p_ref[1] + t0_ref[...]) * dinv + bg0_ref[...], 0.0)
    emb0_ref[...] = emb0
    t1_ref[...] = jnp.dot(
        emb0, wg1_ref[...], preferred_element_type=jnp.float32) * dinv


def _tc_c_body(p_ref, t1_ref, cnt_ref, bg1_ref, emb1_ref):
    dinv = _dinv_of(cnt_ref[...])
    emb1_ref[...] = jnp.maximum(
        (p_ref[0] + p_ref[1] + t1_ref[...]) * dinv + bg1_ref[...], 0.0)


RBD = 2000


def _tc_d_body(hh_ref, wpost_ref, bpost_ref, sc_ref):
    logits = jnp.dot(hh_ref[...], wpost_ref[...]) + bpost_ref[...]
    sc_ref[...] = jnp.sum(logits, axis=-1)


_tc_a = pl.pallas_call(
    _tc_a_body,
    grid=(GRID,),
    in_specs=[_row_spec, _w_spec, _b_spec, _w_spec, _cnt_spec],
    out_specs=_row_spec,
    out_shape=jax.ShapeDtypeStruct((N, H), jnp.float32),
)

_tc_b = pl.pallas_call(
    _tc_b_body,
    grid=(GRID,),
    in_specs=[_p_spec, _row_spec, _cnt_spec, _b_spec, _w_spec],
    out_specs=[_row_spec, _row_spec],
    out_shape=[jax.ShapeDtypeStruct((N, H), jnp.float32),
               jax.ShapeDtypeStruct((N, H), jnp.float32)],
)

_tc_c = pl.pallas_call(
    _tc_c_body,
    grid=(GRID,),
    in_specs=[_p_spec, _row_spec, _cnt_spec, _b_spec],
    out_specs=_row_spec,
    out_shape=jax.ShapeDtypeStruct((N, H), jnp.float32),
)

_tc_d = pl.pallas_call(
    _tc_d_body,
    out_shape=jax.ShapeDtypeStruct((EL_OUT,), jnp.float32),
)


def kernel(x, edge_index, edge_label_index, W_pre, b_pre, W_g0, b_g0,
           W_g1, b_g1, W_post, b_post):
    f32 = jnp.float32
    src = edge_index[0]
    dst = edge_index[1]
    pad = jnp.arange(EL_IDX - EL, dtype=edge_label_index.dtype) % N
    els = jnp.concatenate([edge_label_index[0], pad])
    eld = jnp.concatenate([edge_label_index[1], pad])
    zeros_nh = jnp.zeros((N, H), f32)
    b_pre2 = b_pre.reshape(1, H)
    bg02 = b_g0.reshape(1, H)
    bg12 = b_g1.reshape(1, H)

    cnt = _deg_kernel(dst)
    cnt_t = cnt[:, :N].T

    t0 = _tc_a(x, W_pre, b_pre2, W_g0, cnt_t)
    p0 = _agg_kernel(t0, src, dst, zeros_nh)
    emb0, t1 = _tc_b(p0, t0, cnt_t, bg02, W_g1)
    p1 = _agg_kernel(t1, src, dst, zeros_nh)
    emb1 = _tc_c(p1, t1, cnt_t, bg12)
    hh = _dec_kernel(emb1, els, eld)
    scores = _tc_d(hh, W_post, b_post.reshape(1, 2))[:EL]
    return scores, emb0, emb1
